# baseline (speedup 1.0000x reference)
import functools

import jax
import jax.numpy as jnp
from jax import lax
from jax.experimental import pallas as pl
from jax.experimental.pallas import tpu as pltpu
from jax.experimental.pallas import tpu_sc as plsc

B_BLK = 512
M_GRP = 32

_NC = 2
_NS = 16
_NW = _NC * _NS
_CHUNK = 128
_GRP = 4


def _assign_kernel(vecs_ref, cb_ref, idx_ref):
    K = cb_ref.shape[1]
    dsub = cb_ref.shape[2]
    mg = pl.program_id(0)
    cbs = [cb_ref[j] for j in range(M_GRP)]
    crosses = [
        jnp.dot(vecs_ref[:, j * dsub:(j + 1) * dsub], cbs[j].T,
                preferred_element_type=jnp.float32)
        for j in range(M_GRP)
    ]
    negk = ((K - 1) - lax.broadcasted_iota(
        jnp.int32, (B_BLK, K), 1)).astype(jnp.float32)
    cols = []
    for j in range(M_GRP):
        c_sq_half = 0.5 * jnp.sum(cbs[j] * cbs[j], axis=1)
        s = crosses[j] - c_sq_half[None, :]
        m = jnp.max(s, axis=1, keepdims=True)
        rk = jnp.where(s == m, negk, -1.0)
        best_rk = jnp.max(rk, axis=1, keepdims=True)
        k_best = (K - 1) - best_rk.astype(jnp.int32)
        cols.append((mg * M_GRP + j) * K + k_best)
    idx_ref[:, :] = jnp.concatenate(cols, axis=1)


def _make_lookup(n_rows, n_cb_rows, dsub):
    n_chunks = n_rows // _CHUNK
    per_w = n_chunks // _NW
    mesh = plsc.VectorSubcoreMesh(core_axis_name="c", subcore_axis_name="s")

    @functools.partial(
        pl.kernel,
        out_type=jax.ShapeDtypeStruct((n_chunks, _CHUNK, dsub), jnp.float32),
        mesh=mesh,
        scratch_types=[
            pltpu.VMEM((per_w, _CHUNK), jnp.int32),
            pltpu.VMEM((_GRP, _CHUNK, dsub), jnp.float32),
            pltpu.SemaphoreType.DMA,
        ],
        compiler_params=pltpu.CompilerParams(use_tc_tiling_on_sc=False),
    )
    def _lookup(cb_hbm, idx_hbm, out_hbm, idx_v, rows_v, sem):
        wid = lax.axis_index("s") * _NC + lax.axis_index("c")
        base = wid * per_w
        pltpu.sync_copy(idx_hbm.at[pl.ds(base, per_w)], idx_v)

        def body(g, carry):
            copies = [
                pltpu.async_copy(
                    cb_hbm.at[idx_v.at[g * _GRP + b]], rows_v.at[b], sem)
                for b in range(_GRP)
            ]
            for cp in copies:
                cp.wait()
            pltpu.sync_copy(rows_v, out_hbm.at[pl.ds(base + g * _GRP, _GRP)])
            return carry

        lax.fori_loop(0, per_w // _GRP, body, 0)

    return _lookup


_NSPLIT = 2


def kernel(vecs, codebook):
    B, D = vecs.shape
    M, K, dsub = codebook.shape
    cb_flat = codebook.reshape(M * K, dsub)
    bc = B // _NSPLIT

    def assign(v):
        return pl.pallas_call(
            _assign_kernel,
            grid=(M // M_GRP, bc // B_BLK),
            in_specs=[
                pl.BlockSpec((B_BLK, M_GRP * dsub), lambda mg, i: (i, mg)),
                pl.BlockSpec((M_GRP, K, dsub), lambda mg, i: (mg, 0, 0)),
            ],
            out_specs=pl.BlockSpec((B_BLK, M_GRP), lambda mg, i: (i, mg)),
            out_shape=jax.ShapeDtypeStruct((bc, M), jnp.int32),
            compiler_params=pltpu.CompilerParams(
                dimension_semantics=("parallel", "parallel"),
            ),
        )(v, codebook)

    lookup = _make_lookup(bc * M, M * K, dsub)
    idxs = [assign(vecs[g * bc:(g + 1) * bc]) for g in range(_NSPLIT)]
    rows = [lookup(cb_flat, ix.reshape(-1, _CHUNK)) for ix in idxs]
    return jnp.concatenate([r.reshape(bc, D) for r in rows], axis=0)


# speedup vs baseline: 1.0936x; 1.0936x over previous
import functools

import jax
import jax.numpy as jnp
from jax import lax
from jax.experimental import pallas as pl
from jax.experimental.pallas import tpu as pltpu
from jax.experimental.pallas import tpu_sc as plsc

B_BLK = 512
M_GRP = 32

_NC = 2
_NS = 16
_NW = _NC * _NS
_CHUNK = 128
_GRP = 4


def _assign_kernel(vecs_ref, cb_ref, idx_ref):
    K = cb_ref.shape[1]
    dsub = cb_ref.shape[2]
    mg = pl.program_id(0)
    cbs = [cb_ref[j] for j in range(M_GRP)]
    crosses = [
        jnp.dot(vecs_ref[:, j * dsub:(j + 1) * dsub], cbs[j].T,
                preferred_element_type=jnp.float32)
        for j in range(M_GRP)
    ]
    negk = ((K - 1) - lax.broadcasted_iota(
        jnp.int32, (B_BLK, K), 1)).astype(jnp.float32)
    cols = []
    for j in range(M_GRP):
        c_sq_half = 0.5 * jnp.sum(cbs[j] * cbs[j], axis=1)
        s = crosses[j] - c_sq_half[None, :]
        m = jnp.max(s, axis=1, keepdims=True)
        rk = jnp.where(s == m, negk, -1.0)
        best_rk = jnp.max(rk, axis=1, keepdims=True)
        k_best = (K - 1) - best_rk.astype(jnp.int32)
        cols.append((mg * M_GRP + j) * K + k_best)
    idx_ref[:, :] = jnp.concatenate(cols, axis=1)


def _make_lookup(n_rows, n_cb_rows, dsub):
    n_chunks = n_rows // _CHUNK
    per_w = n_chunks // _NW
    mesh = plsc.VectorSubcoreMesh(core_axis_name="c", subcore_axis_name="s")

    @functools.partial(
        pl.kernel,
        out_type=jax.ShapeDtypeStruct((n_chunks, _CHUNK, dsub), jnp.float32),
        mesh=mesh,
        scratch_types=[
            pltpu.VMEM((per_w, _CHUNK), jnp.int32),
            pltpu.VMEM((_GRP, _CHUNK, dsub), jnp.float32),
            pltpu.SemaphoreType.DMA,
        ],
        compiler_params=pltpu.CompilerParams(use_tc_tiling_on_sc=False),
    )
    def _lookup(cb_hbm, idx_hbm, out_hbm, idx_v, rows_v, sem):
        wid = lax.axis_index("s") * _NC + lax.axis_index("c")
        base = wid * per_w
        pltpu.sync_copy(idx_hbm.at[pl.ds(base, per_w)], idx_v)

        def body(g, carry):
            copies = [
                pltpu.async_copy(
                    cb_hbm.at[idx_v.at[g * _GRP + b]], rows_v.at[b], sem)
                for b in range(_GRP)
            ]
            for cp in copies:
                cp.wait()
            pltpu.sync_copy(rows_v, out_hbm.at[pl.ds(base + g * _GRP, _GRP)])
            return carry

        lax.fori_loop(0, per_w // _GRP, body, 0)

    return _lookup


_NSPLIT = 1


def kernel(vecs, codebook):
    B, D = vecs.shape
    M, K, dsub = codebook.shape
    cb_flat = codebook.reshape(M * K, dsub)
    bc = B // _NSPLIT

    def assign(v):
        return pl.pallas_call(
            _assign_kernel,
            grid=(M // M_GRP, bc // B_BLK),
            in_specs=[
                pl.BlockSpec((B_BLK, M_GRP * dsub), lambda mg, i: (i, mg)),
                pl.BlockSpec((M_GRP, K, dsub), lambda mg, i: (mg, 0, 0)),
            ],
            out_specs=pl.BlockSpec((B_BLK, M_GRP), lambda mg, i: (i, mg)),
            out_shape=jax.ShapeDtypeStruct((bc, M), jnp.int32),
            compiler_params=pltpu.CompilerParams(
                dimension_semantics=("parallel", "parallel"),
            ),
        )(v, codebook)

    lookup = _make_lookup(bc * M, M * K, dsub)
    idxs = [assign(vecs[g * bc:(g + 1) * bc]) for g in range(_NSPLIT)]
    rows = [lookup(cb_flat, ix.reshape(-1, _CHUNK)) for ix in idxs]
    return jnp.concatenate([r.reshape(bc, D) for r in rows], axis=0)


# speedup vs baseline: 1.2143x; 1.1104x over previous
import functools

import jax
import jax.numpy as jnp
from jax import lax
from jax.experimental import pallas as pl
from jax.experimental.pallas import tpu as pltpu
from jax.experimental.pallas import tpu_sc as plsc

B_BLK = 512
M_GRP = 32

_NC = 2
_NS = 16
_NW = _NC * _NS
_CHUNK = 128
_GRP = 4


def _assign_kernel(vecs_ref, cb_ref, idx_ref):
    K = cb_ref.shape[1]
    dsub = cb_ref.shape[2]
    mg = pl.program_id(0)
    cbs = [cb_ref[j] for j in range(M_GRP)]
    ones = jnp.ones((B_BLK, 1), jnp.float32)
    scores = []
    for j in range(M_GRP):
        c_row = (-0.5 * jnp.sum(cbs[j] * cbs[j], axis=1))[None, :]
        cb_aug = jnp.concatenate([cbs[j].T, c_row], axis=0)
        v_aug = jnp.concatenate(
            [vecs_ref[:, j * dsub:(j + 1) * dsub], ones], axis=1)
        scores.append(jnp.dot(v_aug, cb_aug,
                              preferred_element_type=jnp.float32))
    negk = ((K - 1) - lax.broadcasted_iota(
        jnp.int32, (B_BLK, K), 1)).astype(jnp.float32)
    cols = []
    for j in range(M_GRP):
        s = scores[j]
        m = jnp.max(s, axis=1, keepdims=True)
        rk = jnp.where(s == m, negk, -1.0)
        best_rk = jnp.max(rk, axis=1, keepdims=True)
        k_best = (K - 1) - best_rk.astype(jnp.int32)
        cols.append((mg * M_GRP + j) * K + k_best)
    idx_ref[:, :] = jnp.concatenate(cols, axis=1)


def _make_lookup(n_rows, n_cb_rows, dsub):
    n_chunks = n_rows // _CHUNK
    per_w = n_chunks // _NW
    mesh = plsc.VectorSubcoreMesh(core_axis_name="c", subcore_axis_name="s")

    @functools.partial(
        pl.kernel,
        out_type=jax.ShapeDtypeStruct((n_chunks, _CHUNK, dsub), jnp.float32),
        mesh=mesh,
        scratch_types=[
            pltpu.VMEM((per_w, _CHUNK), jnp.int32),
            pltpu.VMEM((_GRP, _CHUNK, dsub), jnp.float32),
            pltpu.SemaphoreType.DMA,
        ],
        compiler_params=pltpu.CompilerParams(use_tc_tiling_on_sc=False),
    )
    def _lookup(cb_hbm, idx_hbm, out_hbm, idx_v, rows_v, sem):
        wid = lax.axis_index("s") * _NC + lax.axis_index("c")
        base = wid * per_w
        pltpu.sync_copy(idx_hbm.at[pl.ds(base, per_w)], idx_v)

        def body(g, carry):
            copies = [
                pltpu.async_copy(
                    cb_hbm.at[idx_v.at[g * _GRP + b]], rows_v.at[b], sem)
                for b in range(_GRP)
            ]
            for cp in copies:
                cp.wait()
            pltpu.sync_copy(rows_v, out_hbm.at[pl.ds(base + g * _GRP, _GRP)])
            return carry

        lax.fori_loop(0, per_w // _GRP, body, 0)

    return _lookup


_NSPLIT = 1


def kernel(vecs, codebook):
    B, D = vecs.shape
    M, K, dsub = codebook.shape
    cb_flat = codebook.reshape(M * K, dsub)
    bc = B // _NSPLIT

    def assign(v):
        return pl.pallas_call(
            _assign_kernel,
            grid=(M // M_GRP, bc // B_BLK),
            in_specs=[
                pl.BlockSpec((B_BLK, M_GRP * dsub), lambda mg, i: (i, mg)),
                pl.BlockSpec((M_GRP, K, dsub), lambda mg, i: (mg, 0, 0)),
            ],
            out_specs=pl.BlockSpec((B_BLK, M_GRP), lambda mg, i: (i, mg)),
            out_shape=jax.ShapeDtypeStruct((bc, M), jnp.int32),
            compiler_params=pltpu.CompilerParams(
                dimension_semantics=("parallel", "parallel"),
            ),
        )(v, codebook)

    lookup = _make_lookup(bc * M, M * K, dsub)
    idxs = [assign(vecs[g * bc:(g + 1) * bc]) for g in range(_NSPLIT)]
    rows = [lookup(cb_flat, ix.reshape(-1, _CHUNK)) for ix in idxs]
    return jnp.concatenate([r.reshape(bc, D) for r in rows], axis=0)
